# baseline (speedup 1.0000x reference)
import jax
import jax.numpy as jnp
from jax import lax
from jax.experimental import pallas as pl
from jax.experimental.pallas import tpu as pltpu
from jax.experimental.pallas import tpu_sc as plsc

_VOCAB = 1000000
_HIDDEN = 128
_BATCH = 16384

_NC = 2
_NS = 16
_NW = _NC * _NS
_B_PER_W = _BATCH // _NW


def _gather_body(table_hbm, idx_hbm, hid_hbm, out_flat, hid_out,
                 idx_v, rows_v, gsems, wsem):
    out_hbm = out_flat.reshape(_BATCH, _HIDDEN)
    wid = lax.axis_index("s") * _NC + lax.axis_index("c")
    base = wid * _B_PER_W
    half = _B_PER_W // 2
    pltpu.sync_copy(idx_hbm.at[pl.ds(base, _B_PER_W)], idx_v)
    g0 = pltpu.async_copy(
        table_hbm.at[idx_v.at[pl.ds(0, half)]], rows_v.at[pl.ds(0, half)],
        gsems.at[0],
    )
    g1 = pltpu.async_copy(
        table_hbm.at[idx_v.at[pl.ds(half, half)]],
        rows_v.at[pl.ds(half, half)],
        gsems.at[1],
    )

    @pl.when(wid == 0)
    def _():
        pltpu.sync_copy(hid_hbm, hid_out)

    g0.wait()
    w0 = pltpu.async_copy(
        rows_v.at[pl.ds(0, half)], out_hbm.at[pl.ds(base, half)], wsem
    )
    g1.wait()
    w1 = pltpu.async_copy(
        rows_v.at[pl.ds(half, half)], out_hbm.at[pl.ds(base + half, half)], wsem
    )
    w0.wait()
    w1.wait()


@jax.jit
def _gather(table, idx, hidden):
    mesh = plsc.VectorSubcoreMesh(core_axis_name="c", subcore_axis_name="s")
    return pl.kernel(
        _gather_body,
        out_type=(
            jax.ShapeDtypeStruct((1, 1, _BATCH * _HIDDEN), jnp.float32),
            jax.ShapeDtypeStruct((1, 1, _HIDDEN), jnp.float32),
        ),
        mesh=mesh,
        scratch_types=[
            pltpu.VMEM((_B_PER_W,), jnp.int32),
            pltpu.VMEM((_B_PER_W, _HIDDEN), jnp.float32),
            pltpu.SemaphoreType.DMA((2,)),
            pltpu.SemaphoreType.DMA,
        ],
    )(table, idx, hidden)


def kernel(input, hidden, embedding):
    idx = input.astype(jnp.int32)
    out, hid = _gather(embedding, idx, hidden)
    return (out, hid)


# speedup vs baseline: 1.0040x; 1.0040x over previous
import jax
import jax.numpy as jnp
from jax import lax
from jax.experimental import pallas as pl
from jax.experimental.pallas import tpu as pltpu
from jax.experimental.pallas import tpu_sc as plsc

_VOCAB = 1000000
_HIDDEN = 128
_BATCH = 16384

_NC = 2
_NS = 16
_NW = _NC * _NS
_B_PER_W = _BATCH // _NW


def _gather_body(table_hbm, idx_hbm, hid_hbm, out_flat, hid_out,
                 idx_v, rows_v, gsem):
    out_hbm = out_flat.reshape(_BATCH, _HIDDEN)
    wid = lax.axis_index("s") * _NC + lax.axis_index("c")
    base = wid * _B_PER_W
    pltpu.sync_copy(idx_hbm.at[pl.ds(base, _B_PER_W)], idx_v)
    gather = pltpu.async_copy(table_hbm.at[idx_v], rows_v, gsem)

    @pl.when(wid == 0)
    def _():
        pltpu.sync_copy(hid_hbm, hid_out)

    gather.wait()
    pltpu.sync_copy(rows_v, out_hbm.at[pl.ds(base, _B_PER_W)])


@jax.jit
def _gather(table, idx, hidden):
    mesh = plsc.VectorSubcoreMesh(core_axis_name="c", subcore_axis_name="s")
    return pl.kernel(
        _gather_body,
        out_type=(
            jax.ShapeDtypeStruct((1, 1, _BATCH * _HIDDEN), jnp.float32),
            jax.ShapeDtypeStruct((1, 1, _HIDDEN), jnp.float32),
        ),
        mesh=mesh,
        scratch_types=[
            pltpu.VMEM((_B_PER_W,), jnp.int32),
            pltpu.VMEM((_B_PER_W, _HIDDEN), jnp.float32),
            pltpu.SemaphoreType.DMA,
        ],
    )(table, idx, hidden)


def kernel(input, hidden, embedding):
    idx = input.astype(jnp.int32)
    out, hid = _gather(embedding, idx, hidden)
    return (out, hid)
